# baseline (speedup 1.0000x reference)
import functools

import jax
import jax.numpy as jnp
from jax import lax
from jax.experimental import pallas as pl
from jax.experimental.pallas import tpu as pltpu
from jax.experimental.pallas import tpu_sc as plsc

EPS = 1e-12
L = 16


def _tile_linear_1d(x):
    b, n = x.shape
    return x.reshape(b // 128, 128, n // 8, 8).transpose(2, 0, 3, 1).reshape(b * n)


def kernel(P_vocab, attn_dist, p_gen, target_idx, copy_idx):
    B, V = P_vocab.shape
    S = attn_dist.shape[1]

    pv1 = _tile_linear_1d(P_vocab)
    at1 = _tile_linear_1d(attn_dist)
    pg1 = p_gen.reshape(B)
    tg = target_idx.astype(jnp.int32)
    cp = copy_idx.astype(jnp.int32)

    info = plsc.get_sparse_core_info()
    NC = 1
    NS = info.num_subcores
    NW = NC * NS
    b_per_w = B // NW
    n_chunks = b_per_w // L

    mesh = plsc.VectorSubcoreMesh(core_axis_name="c", subcore_axis_name="s", num_cores=1)

    @functools.partial(
        pl.kernel,
        mesh=mesh,
        compiler_params=pltpu.CompilerParams(needs_layout_passes=False),
        out_type=jax.ShapeDtypeStruct((B,), jnp.float32),
        scratch_types=[
            pltpu.VMEM((b_per_w,), jnp.int32),
            pltpu.VMEM((b_per_w,), jnp.int32),
            pltpu.VMEM((b_per_w,), jnp.float32),
            pltpu.VMEM((b_per_w,), jnp.int32),
            pltpu.VMEM((b_per_w,), jnp.int32),
            pltpu.VMEM((b_per_w,), jnp.float32),
            pltpu.VMEM((b_per_w,), jnp.float32),
            pltpu.VMEM((b_per_w,), jnp.float32),
            pltpu.SemaphoreType.DMA,
            pltpu.SemaphoreType.DMA,
        ],
    )
    def gather_k(pv_hbm, at_hbm, pg_hbm, tg_hbm, cp_hbm, out_hbm,
                 tg_v, cp_v, pg_v, pidx_v, aidx_v, psel_v, asel_v,
                 out_v, sem_p, sem_a):
        wid = lax.axis_index("s") * NC + lax.axis_index("c")
        base = wid * b_per_w
        pltpu.sync_copy(tg_hbm.at[pl.ds(base, b_per_w)], tg_v)
        pltpu.sync_copy(cp_hbm.at[pl.ds(base, b_per_w)], cp_v)
        pltpu.sync_copy(pg_hbm.at[pl.ds(base, b_per_w)], pg_v)

        iot = lax.iota(jnp.int32, L)
        for j in range(n_chunks):
            sl = pl.ds(j * L, L)
            bvec = base + j * L + iot
            t = jnp.minimum(jnp.maximum(tg_v[sl], 0), V - 1)
            c = jnp.minimum(jnp.maximum(cp_v[sl], 0), S - 1)
            bpart = (lax.shift_right_logical(bvec, 7) * 1024
                     + (bvec & 127))
            pidx_v[sl] = (lax.shift_right_logical(t, 3) * 8192
                          + (t & 7) * 128 + bpart)
            aidx_v[sl] = (lax.shift_right_logical(c, 3) * 8192
                          + (c & 7) * 128 + bpart)

        cp_p = pltpu.async_copy(pv_hbm.at[pidx_v], psel_v, sem_p)
        cp_a = pltpu.async_copy(at_hbm.at[aidx_v], asel_v, sem_a)
        cp_p.wait()
        cp_a.wait()

        for j in range(n_chunks):
            sl = pl.ds(j * L, L)
            g = tg_v[sl] < V
            pgv = pg_v[sl]
            out_v[sl] = jnp.where(g, pgv * psel_v[sl] + EPS,
                                  (1.0 - pgv) * asel_v[sl] + EPS)

        pltpu.sync_copy(out_v, out_hbm.at[pl.ds(base, b_per_w)])

    arg = gather_k(pv1, at1, pg1, tg, cp)

    def loss_body(x_ref, o_ref):
        o_ref[0, 0] = -jnp.sum(jnp.log(x_ref[...])) * (1.0 / B)

    loss = pl.pallas_call(
        loss_body,
        out_shape=jax.ShapeDtypeStruct((1, 1), jnp.float32),
        out_specs=pl.BlockSpec(memory_space=pltpu.SMEM),
    )(arg.reshape(B // 128, 128))
    return loss[0, 0]


# speedup vs baseline: 1.0547x; 1.0547x over previous
import functools

import jax
import jax.numpy as jnp
from jax import lax
from jax.experimental import pallas as pl
from jax.experimental.pallas import tpu as pltpu
from jax.experimental.pallas import tpu_sc as plsc

EPS = 1e-12
L = 16
LN2 = 0.6931471805599453
SQRT2 = 1.4142135381698608


def _tile_linear_1d(x):
    b, n = x.shape
    return x.reshape(b // 128, 128, n // 8, 8).transpose(2, 0, 3, 1).reshape(b * n)


def kernel(P_vocab, attn_dist, p_gen, target_idx, copy_idx):
    B, V = P_vocab.shape
    S = attn_dist.shape[1]

    pv1 = _tile_linear_1d(P_vocab)
    at1 = _tile_linear_1d(attn_dist)
    pg1 = p_gen.reshape(B)
    tg = target_idx.astype(jnp.int32)
    cp = copy_idx.astype(jnp.int32)

    info = plsc.get_sparse_core_info()
    NW = info.num_subcores
    b_per_w = B // NW
    n_chunks = b_per_w // L

    mesh = plsc.VectorSubcoreMesh(core_axis_name="c", subcore_axis_name="s",
                                  num_cores=1)

    @functools.partial(
        pl.kernel,
        mesh=mesh,
        compiler_params=pltpu.CompilerParams(needs_layout_passes=False),
        out_type=jax.ShapeDtypeStruct((L,), jnp.float32),
        scratch_types=[
            pltpu.VMEM((b_per_w,), jnp.int32),
            pltpu.VMEM((b_per_w,), jnp.int32),
            pltpu.VMEM((b_per_w,), jnp.float32),
            pltpu.VMEM((b_per_w,), jnp.int32),
            pltpu.VMEM((b_per_w,), jnp.int32),
            pltpu.VMEM((b_per_w,), jnp.float32),
            pltpu.VMEM((b_per_w,), jnp.float32),
            pltpu.VMEM((L,), jnp.float32),
            pltpu.VMEM((NW * L,), jnp.float32),
            pltpu.VMEM((L,), jnp.float32),
            pltpu.VMEM_SHARED((NW * L,), jnp.float32),
            pltpu.SemaphoreType.DMA,
            pltpu.SemaphoreType.DMA,
        ],
    )
    def gather_k(pv_hbm, at_hbm, pg_hbm, tg_hbm, cp_hbm, out_hbm,
                 tg_v, cp_v, pg_v, pidx_v, aidx_v, psel_v, asel_v,
                 part_v, accbuf, outbuf, shared, sem_p, sem_a):
        wid = lax.axis_index("s")
        base = wid * b_per_w
        pltpu.sync_copy(tg_hbm.at[pl.ds(base, b_per_w)], tg_v)
        pltpu.sync_copy(cp_hbm.at[pl.ds(base, b_per_w)], cp_v)
        pltpu.sync_copy(pg_hbm.at[pl.ds(base, b_per_w)], pg_v)

        iot = lax.iota(jnp.int32, L)
        for j in range(n_chunks):
            sl = pl.ds(j * L, L)
            bvec = base + j * L + iot
            t = jnp.minimum(jnp.maximum(tg_v[sl], 0), V - 1)
            c = jnp.minimum(jnp.maximum(cp_v[sl], 0), S - 1)
            bpart = lax.shift_right_logical(bvec, 7) * 1024 + (bvec & 127)
            pidx_v[sl] = (lax.shift_right_logical(t, 3) * 8192
                          + (t & 7) * 128 + bpart)
            aidx_v[sl] = (lax.shift_right_logical(c, 3) * 8192
                          + (c & 7) * 128 + bpart)

        cp_p = pltpu.async_copy(pv_hbm.at[pidx_v], psel_v, sem_p)
        cp_a = pltpu.async_copy(at_hbm.at[aidx_v], asel_v, sem_a)
        cp_p.wait()
        cp_a.wait()

        acc = iot.astype(jnp.float32) * 0.0
        for j in range(n_chunks):
            sl = pl.ds(j * L, L)
            g = tg_v[sl] < V
            pgv = pg_v[sl]
            x = jnp.where(g, pgv * psel_v[sl] + EPS,
                          (1.0 - pgv) * asel_v[sl] + EPS)
            bits = plsc.bitcast(x, jnp.int32)
            e = lax.shift_right_arithmetic(bits, 23) - 127
            m = plsc.bitcast((bits & 0x7FFFFF) | 0x3F800000, jnp.float32)
            big = m > SQRT2
            m = jnp.where(big, m * 0.5, m)
            ef = (e + big.astype(jnp.int32)).astype(jnp.float32)
            s = (m - 1.0) / (m + 1.0)
            z = s * s
            p = (1.0 / 9.0) * z + (1.0 / 7.0)
            p = p * z + 0.2
            p = p * z + (1.0 / 3.0)
            p = p * z + 1.0
            acc = acc + ef * LN2 + 2.0 * s * p
        part_v[pl.ds(0, L)] = acc
        pltpu.sync_copy(part_v, shared.at[pl.ds(wid * L, L)])
        plsc.subcore_barrier()

        @pl.when(wid == 0)
        def _():
            pltpu.sync_copy(shared, accbuf)
            tot = accbuf[pl.ds(0, L)]
            for w in range(1, NW):
                tot = tot + accbuf[pl.ds(w * L, L)]
            r = jnp.sum(tot) * (-1.0 / B)
            outbuf[pl.ds(0, L)] = r + iot.astype(jnp.float32) * 0.0
            pltpu.sync_copy(outbuf, out_hbm)

    loss = gather_k(pv1, at1, pg1, tg, cp)
    return loss[0]


# speedup vs baseline: 1.0947x; 1.0379x over previous
import functools

import jax
import jax.numpy as jnp
from jax import lax
from jax.experimental import pallas as pl
from jax.experimental.pallas import tpu as pltpu
from jax.experimental.pallas import tpu_sc as plsc

EPS = 1e-12
L = 16
LN2 = 0.6931471805599453
SQRT2 = 1.4142135381698608


def _tile_linear_1d(x):
    b, n = x.shape
    return x.reshape(b // 128, 128, n // 8, 8).transpose(2, 0, 3, 1).reshape(b * n)


def kernel(P_vocab, attn_dist, p_gen, target_idx, copy_idx):
    B, V = P_vocab.shape
    S = attn_dist.shape[1]

    pv1 = _tile_linear_1d(P_vocab)
    at1 = _tile_linear_1d(attn_dist)
    pg1 = p_gen.reshape(B)
    tg = target_idx.astype(jnp.int32)
    cp = copy_idx.astype(jnp.int32)

    info = plsc.get_sparse_core_info()
    NW = info.num_subcores
    b_per_w = B // NW
    n_chunks = b_per_w // L

    mesh = plsc.VectorSubcoreMesh(core_axis_name="c", subcore_axis_name="s",
                                  num_cores=1)

    @functools.partial(
        pl.kernel,
        mesh=mesh,
        compiler_params=pltpu.CompilerParams(needs_layout_passes=False),
        out_type=jax.ShapeDtypeStruct((L,), jnp.float32),
        scratch_types=[
            pltpu.VMEM((b_per_w,), jnp.int32),
            pltpu.VMEM((b_per_w,), jnp.int32),
            pltpu.VMEM((b_per_w,), jnp.float32),
            pltpu.VMEM((b_per_w,), jnp.int32),
            pltpu.VMEM((b_per_w,), jnp.int32),
            pltpu.VMEM((b_per_w,), jnp.float32),
            pltpu.VMEM((b_per_w,), jnp.float32),
            pltpu.VMEM((L,), jnp.float32),
            pltpu.VMEM((NW * L,), jnp.float32),
            pltpu.VMEM((L,), jnp.float32),
            pltpu.VMEM_SHARED((NW * L,), jnp.float32),
            pltpu.SemaphoreType.DMA,
            pltpu.SemaphoreType.DMA,
        ],
    )
    def gather_k(pv_hbm, at_hbm, pg_hbm, tg_hbm, cp_hbm, out_hbm,
                 tg_v, cp_v, pg_v, pidx_v, aidx_v, psel_v, asel_v,
                 part_v, accbuf, outbuf, shared, sem_p, sem_a):
        wid = lax.axis_index("s")
        base = wid * b_per_w
        w1 = pltpu.async_copy(tg_hbm.at[pl.ds(base, b_per_w)], tg_v, sem_p)
        w2 = pltpu.async_copy(cp_hbm.at[pl.ds(base, b_per_w)], cp_v, sem_a)
        w3 = pltpu.async_copy(pg_hbm.at[pl.ds(base, b_per_w)], pg_v, sem_p)
        w1.wait()
        w2.wait()
        w3.wait()

        iot = lax.iota(jnp.int32, L)
        for j in range(n_chunks):
            sl = pl.ds(j * L, L)
            bvec = base + j * L + iot
            t = jnp.minimum(jnp.maximum(tg_v[sl], 0), V - 1)
            c = jnp.minimum(jnp.maximum(cp_v[sl], 0), S - 1)
            bpart = lax.shift_right_logical(bvec, 7) * 1024 + (bvec & 127)
            pidx_v[sl] = (lax.shift_right_logical(t, 3) * 8192
                          + (t & 7) * 128 + bpart)
            aidx_v[sl] = (lax.shift_right_logical(c, 3) * 8192
                          + (c & 7) * 128 + bpart)

        cp_p = pltpu.async_copy(pv_hbm.at[pidx_v], psel_v, sem_p)
        cp_a = pltpu.async_copy(at_hbm.at[aidx_v], asel_v, sem_a)
        cp_p.wait()
        cp_a.wait()

        acc = iot.astype(jnp.float32) * 0.0
        for j in range(n_chunks):
            sl = pl.ds(j * L, L)
            g = tg_v[sl] < V
            pgv = pg_v[sl]
            x = jnp.where(g, pgv * psel_v[sl] + EPS,
                          (1.0 - pgv) * asel_v[sl] + EPS)
            bits = plsc.bitcast(x, jnp.int32)
            e = lax.shift_right_arithmetic(bits, 23) - 127
            m = plsc.bitcast((bits & 0x7FFFFF) | 0x3F800000, jnp.float32)
            big = m > SQRT2
            m = jnp.where(big, m * 0.5, m)
            ef = (e + big.astype(jnp.int32)).astype(jnp.float32)
            s = (m - 1.0) / (m + 1.0)
            z = s * s
            p = (1.0 / 9.0) * z + (1.0 / 7.0)
            p = p * z + 0.2
            p = p * z + (1.0 / 3.0)
            p = p * z + 1.0
            acc = acc + ef * LN2 + 2.0 * s * p
        part_v[pl.ds(0, L)] = acc
        pltpu.sync_copy(part_v, shared.at[pl.ds(wid * L, L)])
        plsc.subcore_barrier()

        @pl.when(wid == 0)
        def _():
            pltpu.sync_copy(shared, accbuf)
            tot = accbuf[pl.ds(0, L)]
            for w in range(1, NW):
                tot = tot + accbuf[pl.ds(w * L, L)]
            r = jnp.sum(tot) * (-1.0 / B)
            outbuf[pl.ds(0, L)] = r + iot.astype(jnp.float32) * 0.0
            pltpu.sync_copy(outbuf, out_hbm)

    loss = gather_k(pv1, at1, pg1, tg, cp)
    return loss[0]
